# baseline (speedup 1.0000x reference)
import functools

import jax
import jax.numpy as jnp
from jax import lax
from jax.experimental import pallas as pl
from jax.experimental.pallas import tpu as pltpu
from jax.experimental.pallas import tpu_sc as plsc

VOCAB = 1000
BATCH = 1024
SEQ = 50
SCH = 16
DIM = VOCAB
DIMP = 1024
ROWSP = VOCAB + 104
MAIN = 896
TAIL = DIM - MAIN

_INFO = plsc.get_sparse_core_info()
NC = _INFO.num_cores
NS = _INFO.num_subcores
NW = NC * NS
BPW = BATCH // NW


def _tail_rows(tail_v, buf, nrows):
  def tail_row(r, carry):
    for i in range(TAIL // 16):
      tail_v[r, pl.ds(i * 16, 16)] = buf[r, pl.ds(MAIN + i * 16, 16)]
    tail_v[r, pl.ds(TAIL - 16, 16)] = buf[r, pl.ds(MAIN + TAIL - 16, 16)]
    return carry

  lax.fori_loop(0, nrows, tail_row, 0)


def _make_sc_gather16():
  mesh = plsc.VectorSubcoreMesh(core_axis_name="c", subcore_axis_name="s")
  NCH = BPW // 2

  @functools.partial(
      pl.kernel,
      mesh=mesh,
      out_type=jax.ShapeDtypeStruct((BATCH, SCH, DIM), jnp.float32),
      scratch_types=[
          pltpu.VMEM((NCH, 2 * SCH), jnp.int32),
          pltpu.VMEM((2 * SCH, DIMP), jnp.float32),
          pltpu.VMEM((2 * SCH, DIMP), jnp.float32),
          pltpu.VMEM((2 * SCH, TAIL), jnp.float32),
          pltpu.SemaphoreType.DMA,
          pltpu.SemaphoreType.DMA,
      ],
      compiler_params=pltpu.CompilerParams(use_tc_tiling_on_sc=True),
  )
  def body(table_hbm, idx_hbm, out_hbm, idx_v, buf0, buf1, tail_v,
           sem0, sem1):
    wid = lax.axis_index("s") * NC + lax.axis_index("c")
    base = wid * BPW

    pltpu.sync_copy(idx_hbm.at[wid], idx_v)

    def gather(c, buf, sem):
      return pltpu.make_async_copy(table_hbm.at[idx_v.at[c]], buf, sem)

    def writeback(c, buf):
      _tail_rows(tail_v, buf, 2 * SCH)
      for k in range(2):
        b = base + 2 * c + k
        pltpu.sync_copy(buf.at[pl.ds(k * SCH, SCH), pl.ds(0, MAIN)],
                        out_hbm.at[b, :, pl.ds(0, MAIN)])
        pltpu.sync_copy(tail_v.at[pl.ds(k * SCH, SCH), :],
                        out_hbm.at[b, :, pl.ds(MAIN, TAIL)])

    gather(0, buf0, sem0).start()
    gather(1, buf1, sem1).start()

    def step(i, carry):
      c0 = 2 * i
      c1 = c0 + 1
      gather(c0, buf0, sem0).wait()
      writeback(c0, buf0)

      @pl.when(c0 + 2 < NCH)
      def _():
        gather(c0 + 2, buf0, sem0).start()

      gather(c1, buf1, sem1).wait()
      writeback(c1, buf1)

      @pl.when(c1 + 2 < NCH)
      def _():
        gather(c1 + 2, buf1, sem1).start()

      return carry

    lax.fori_loop(0, NCH // 2, step, 0)

  return body


def _make_sc_gather2():
  mesh = plsc.VectorSubcoreMesh(core_axis_name="c", subcore_axis_name="s")

  @functools.partial(
      pl.kernel,
      mesh=mesh,
      out_type=jax.ShapeDtypeStruct((BATCH, 2, DIM), jnp.float32),
      scratch_types=[
          pltpu.VMEM((BPW, 2), jnp.int32),
          pltpu.VMEM((2, DIMP), jnp.float32),
          pltpu.VMEM((2, DIMP), jnp.float32),
          pltpu.VMEM((2, TAIL), jnp.float32),
          pltpu.SemaphoreType.DMA,
          pltpu.SemaphoreType.DMA,
      ],
      compiler_params=pltpu.CompilerParams(use_tc_tiling_on_sc=True),
  )
  def body(table_hbm, idx_hbm, out_hbm, idx_v, buf0, buf1, tail_v,
           sem0, sem1):
    wid = lax.axis_index("s") * NC + lax.axis_index("c")
    base = wid * BPW

    pltpu.sync_copy(idx_hbm.at[wid], idx_v)

    def gather(c, buf, sem):
      return pltpu.make_async_copy(table_hbm.at[idx_v.at[c]], buf, sem)

    def writeback(c, buf):
      _tail_rows(tail_v, buf, 2)
      pltpu.sync_copy(buf.at[:, pl.ds(0, MAIN)],
                      out_hbm.at[base + c, :, pl.ds(0, MAIN)])
      pltpu.sync_copy(tail_v, out_hbm.at[base + c, :, pl.ds(MAIN, TAIL)])

    gather(0, buf0, sem0).start()
    gather(1, buf1, sem1).start()

    def step(i, carry):
      c0 = 2 * i
      c1 = c0 + 1
      gather(c0, buf0, sem0).wait()
      writeback(c0, buf0)

      @pl.when(c0 + 2 < BPW)
      def _():
        gather(c0 + 2, buf0, sem0).start()

      gather(c1, buf1, sem1).wait()
      writeback(c1, buf1)

      @pl.when(c1 + 2 < BPW)
      def _():
        gather(c1 + 2, buf1, sem1).start()

      return carry

    lax.fori_loop(0, BPW // 2, step, 0)

  return body


_sc_gather16 = _make_sc_gather16()
_sc_gather2 = _make_sc_gather2()


def kernel(idx, token_embedding_table):
  idx_w = idx.astype(jnp.int32)
  table_p = jnp.pad(token_embedding_table,
                    ((0, ROWSP - VOCAB), (0, DIMP - DIM)))
  parts = []
  for p in range(SEQ // SCH):
    idx_p = idx_w[:, p * SCH:(p + 1) * SCH].reshape(NW, BPW // 2, 2 * SCH)
    parts.append(_sc_gather16(table_p, idx_p))
  idx_t = idx_w[:, SEQ - 2:].reshape(NW, BPW, 2)
  parts.append(_sc_gather2(table_p, idx_t))
  out = jnp.zeros((BATCH, SEQ, DIM), jnp.float32)
  for p, part in enumerate(parts):
    out = lax.dynamic_update_slice(out, part, (0, p * SCH, 0))
  return out


# speedup vs baseline: 1.3096x; 1.3096x over previous
import functools

import jax
import jax.numpy as jnp
from jax import lax
from jax.experimental import pallas as pl
from jax.experimental.pallas import tpu as pltpu
from jax.experimental.pallas import tpu_sc as plsc

VOCAB = 1000
BATCH = 1024
SEQ = 50
SEQA = 48
DIM = VOCAB
DIMP = 1024
ROWSP = VOCAB + 104
MAIN = 896
TAIL = DIM - MAIN

_INFO = plsc.get_sparse_core_info()
NC = _INFO.num_cores
NS = _INFO.num_subcores
NW = NC * NS
BPW = BATCH // NW


def _make_sc_gather():
  mesh = plsc.VectorSubcoreMesh(core_axis_name="c", subcore_axis_name="s")

  @functools.partial(
      pl.kernel,
      mesh=mesh,
      out_type=jax.ShapeDtypeStruct((BATCH, SEQ, DIM), jnp.float32),
      scratch_types=[
          pltpu.VMEM((BPW, SEQA), jnp.int32),
          pltpu.VMEM((BPW, 2), jnp.int32),
          pltpu.VMEM((SEQA, DIMP), jnp.float32),
          pltpu.VMEM((SEQA, DIMP), jnp.float32),
          pltpu.VMEM((2, DIMP), jnp.float32),
          pltpu.VMEM((2, DIMP), jnp.float32),
          pltpu.VMEM((SEQ, TAIL), jnp.float32),
          pltpu.SemaphoreType.DMA,
          pltpu.SemaphoreType.DMA,
          pltpu.SemaphoreType.DMA,
          pltpu.SemaphoreType.DMA,
      ],
      compiler_params=pltpu.CompilerParams(use_tc_tiling_on_sc=True),
  )
  def body(table_hbm, idxa_hbm, idxb_hbm, out_hbm, idxa_v, idxb_v,
           buf0, buf1, sb0, sb1, tail_v, sem0, sem1, semb0, semb1):
    wid = lax.axis_index("s") * NC + lax.axis_index("c")
    base = wid * BPW

    pltpu.sync_copy(idxa_hbm.at[wid], idxa_v)
    pltpu.sync_copy(idxb_hbm.at[wid], idxb_v)

    def gather(c, buf, sem):
      return pltpu.make_async_copy(table_hbm.at[idxa_v.at[c]], buf, sem)

    def gather_b(c, sb, semb):
      return pltpu.make_async_copy(table_hbm.at[idxb_v.at[c]], sb, semb)

    def writeback(c, buf, sb):
      def tail_row(dst, r, src, q):
        for i in range(TAIL // 16):
          dst[r, pl.ds(i * 16, 16)] = src[q, pl.ds(MAIN + i * 16, 16)]
        dst[r, pl.ds(TAIL - 16, 16)] = src[q, pl.ds(MAIN + TAIL - 16, 16)]

      def row(r, carry):
        tail_row(tail_v, r, buf, r)
        return carry

      lax.fori_loop(0, SEQA, row, 0)
      for k in range(SEQ - SEQA):
        tail_row(tail_v, SEQA + k, sb, k)

      pltpu.sync_copy(buf.at[:, pl.ds(0, MAIN)],
                      out_hbm.at[base + c, pl.ds(0, SEQA), pl.ds(0, MAIN)])
      pltpu.sync_copy(sb.at[:, pl.ds(0, MAIN)],
                      out_hbm.at[base + c, pl.ds(SEQA, SEQ - SEQA),
                                 pl.ds(0, MAIN)])
      pltpu.sync_copy(tail_v, out_hbm.at[base + c, :, pl.ds(MAIN, TAIL)])

    gather(0, buf0, sem0).start()
    gather_b(0, sb0, semb0).start()
    gather(1, buf1, sem1).start()
    gather_b(1, sb1, semb1).start()

    def step(i, carry):
      c0 = 2 * i
      c1 = c0 + 1

      gather(c0, buf0, sem0).wait()
      gather_b(c0, sb0, semb0).wait()
      writeback(c0, buf0, sb0)

      @pl.when(c0 + 2 < BPW)
      def _():
        gather(c0 + 2, buf0, sem0).start()
        gather_b(c0 + 2, sb0, semb0).start()

      gather(c1, buf1, sem1).wait()
      gather_b(c1, sb1, semb1).wait()
      writeback(c1, buf1, sb1)

      @pl.when(c1 + 2 < BPW)
      def _():
        gather(c1 + 2, buf1, sem1).start()
        gather_b(c1 + 2, sb1, semb1).start()

      return carry

    lax.fori_loop(0, BPW // 2, step, 0)

  return body


_sc_gather = _make_sc_gather()


def kernel(idx, token_embedding_table):
  idx_w = idx.astype(jnp.int32).reshape(NW, BPW, SEQ)
  idx_a = idx_w[:, :, :SEQA]
  idx_b = idx_w[:, :, SEQA:]
  table_p = jnp.pad(token_embedding_table,
                    ((0, ROWSP - VOCAB), (0, DIMP - DIM)))
  return _sc_gather(table_p, idx_a, idx_b)
